# baseline (speedup 1.0000x reference)
import functools
import math

import jax
import jax.numpy as jnp
from jax import lax
from jax.experimental import pallas as pl
from jax.experimental.pallas import tpu as pltpu
from jax.experimental.pallas import tpu_sc as plsc

D_MODEL = 2048
SCALE = math.sqrt(float(D_MODEL))

_INFO = plsc.get_sparse_core_info()
_NC = _INFO.num_cores
_NS = _INFO.num_subcores
_LANES = _INFO.num_lanes
_NW = _NC * _NS

_CHUNK = 8
_VECS_PER_ROW = D_MODEL // _LANES


def _embed_sc(num_tokens: int):
    rows_per_w = num_tokens // _NW
    n_chunks = rows_per_w // _CHUNK
    mesh = plsc.VectorSubcoreMesh(core_axis_name="c", subcore_axis_name="s")

    @functools.partial(
        pl.kernel,
        mesh=mesh,
        out_type=jax.ShapeDtypeStruct((num_tokens, D_MODEL), jnp.float32),
        scratch_types=[
            pltpu.VMEM((rows_per_w,), jnp.int32),
            pltpu.VMEM((_CHUNK, D_MODEL), jnp.float32),
            pltpu.VMEM((_CHUNK, D_MODEL), jnp.float32),
            pltpu.VMEM((_CHUNK, D_MODEL), jnp.float32),
            pltpu.VMEM((_CHUNK, D_MODEL), jnp.float32),
            pltpu.SemaphoreType.DMA,
            pltpu.SemaphoreType.DMA,
            pltpu.SemaphoreType.DMA,
            pltpu.SemaphoreType.DMA,
        ],
    )
    def k(x_hbm, table_hbm, out_hbm, idx_v, g0, g1, s0, s1,
          gsem0, gsem1, ssem0, ssem1):
        wid = lax.axis_index("s") * _NC + lax.axis_index("c")
        base = wid * rows_per_w
        gbufs = (g0, g1)
        sbufs = (s0, s1)
        gsems = (gsem0, gsem1)
        ssems = (ssem0, ssem1)

        pltpu.sync_copy(x_hbm.at[pl.ds(base, rows_per_w)], idx_v)

        def start_gather(chunk, slot):
            pltpu.async_copy(
                table_hbm.at[idx_v.at[pl.ds(chunk * _CHUNK, _CHUNK)]],
                gbufs[slot], gsems[slot])

        def wait_gather(slot):
            pltpu.make_async_copy(
                table_hbm.at[pl.ds(0, _CHUNK)], gbufs[slot],
                gsems[slot]).wait()

        def start_scatter(chunk, slot):
            pltpu.async_copy(
                sbufs[slot],
                out_hbm.at[pl.ds(base + chunk * _CHUNK, _CHUNK)],
                ssems[slot])

        def wait_scatter(slot):
            pltpu.make_async_copy(
                out_hbm.at[pl.ds(0, _CHUNK)], sbufs[slot],
                ssems[slot]).wait()

        start_gather(0, 0)
        start_gather(1, 1)

        def step(i, carry):
            for slot in range(2):
                chunk = i + slot
                wait_gather(slot)

                @pl.when(chunk >= 2)
                def _():
                    wait_scatter(slot)

                gb = gbufs[slot]
                sb = sbufs[slot]

                def scale_row(r, c2):
                    def scale_vec(v, c3):
                        sl = pl.ds(v * _LANES, _LANES)
                        sb[r, sl] = gb[r, sl] * SCALE
                        return c3
                    return lax.fori_loop(0, _VECS_PER_ROW, scale_vec, c2)

                lax.fori_loop(0, _CHUNK, scale_row, 0)

                @pl.when(chunk + 2 < n_chunks)
                def _():
                    start_gather(chunk + 2, slot)

                start_scatter(chunk, slot)
            return carry

        lax.fori_loop(0, n_chunks // 2, lambda j, c: step(j * 2, c), 0)

        wait_scatter(0)
        wait_scatter(1)

    return k


@jax.jit
def kernel(x, table):
    b, s = x.shape
    xf = x.reshape(-1).astype(jnp.int32)
    out = _embed_sc(b * s)(xf, table)
    return out.reshape(b, s, D_MODEL)


# speedup vs baseline: 2.2901x; 2.2901x over previous
import functools
import math

import jax
import jax.numpy as jnp
from jax import lax
from jax.experimental import pallas as pl
from jax.experimental.pallas import tpu as pltpu
from jax.experimental.pallas import tpu_sc as plsc

D_MODEL = 2048
SCALE = math.sqrt(float(D_MODEL))

_INFO = plsc.get_sparse_core_info()
_NC = _INFO.num_cores
_NS = _INFO.num_subcores
_LANES = _INFO.num_lanes
_NW = _NC * _NS

_CHUNK = 8
_VECS_PER_ROW = D_MODEL // _LANES


def _embed_sc(num_tokens: int):
    rows_per_w = num_tokens // _NW
    n_chunks = rows_per_w // _CHUNK
    mesh = plsc.VectorSubcoreMesh(core_axis_name="c", subcore_axis_name="s")

    @functools.partial(
        pl.kernel,
        mesh=mesh,
        out_type=jax.ShapeDtypeStruct((num_tokens, D_MODEL), jnp.float32),
        scratch_types=[
            pltpu.VMEM((rows_per_w,), jnp.int32),
            pltpu.VMEM((_CHUNK, D_MODEL), jnp.float32),
            pltpu.VMEM((_CHUNK, D_MODEL), jnp.float32),
            pltpu.VMEM((_CHUNK, D_MODEL), jnp.float32),
            pltpu.VMEM((_CHUNK, D_MODEL), jnp.float32),
            pltpu.SemaphoreType.DMA,
            pltpu.SemaphoreType.DMA,
            pltpu.SemaphoreType.DMA,
            pltpu.SemaphoreType.DMA,
        ],
    )
    def k(x_hbm, table_hbm, out_hbm, idx_v, g0, g1, s0, s1,
          gsem0, gsem1, ssem0, ssem1):
        wid = lax.axis_index("s") * _NC + lax.axis_index("c")
        base = wid * rows_per_w
        gbufs = (g0, g1)
        sbufs = (s0, s1)
        gsems = (gsem0, gsem1)
        ssems = (ssem0, ssem1)

        pltpu.sync_copy(x_hbm.at[pl.ds(base, rows_per_w)], idx_v)

        def start_gather(chunk, slot):
            pltpu.async_copy(
                table_hbm.at[idx_v.at[pl.ds(chunk * _CHUNK, _CHUNK)]],
                gbufs[slot], gsems[slot])

        def wait_gather(slot):
            pltpu.make_async_copy(
                table_hbm.at[pl.ds(0, _CHUNK)], gbufs[slot],
                gsems[slot]).wait()

        def start_scatter(chunk, slot):
            pltpu.async_copy(
                sbufs[slot],
                out_hbm.at[pl.ds(base + chunk * _CHUNK, _CHUNK)],
                ssems[slot])

        def wait_scatter(slot):
            pltpu.make_async_copy(
                out_hbm.at[pl.ds(0, _CHUNK)], sbufs[slot],
                ssems[slot]).wait()

        start_gather(0, 0)
        start_gather(1, 1)

        def step(i, carry):
            for slot in range(2):
                chunk = i + slot
                wait_gather(slot)

                @pl.when(chunk >= 2)
                def _():
                    wait_scatter(slot)

                gb = gbufs[slot]
                sb = sbufs[slot]

                def scale_row(r, c2):
                    for v in range(_VECS_PER_ROW):
                        sl = pl.ds(v * _LANES, _LANES)
                        sb[r, sl] = gb[r, sl] * SCALE
                    return c2

                lax.fori_loop(0, _CHUNK, scale_row, 0)

                @pl.when(chunk + 2 < n_chunks)
                def _():
                    start_gather(chunk + 2, slot)

                start_scatter(chunk, slot)
            return carry

        lax.fori_loop(0, n_chunks // 2, lambda j, c: step(j * 2, c), 0)

        wait_scatter(0)
        wait_scatter(1)

    return k


@jax.jit
def kernel(x, table):
    b, s = x.shape
    xf = x.reshape(-1).astype(jnp.int32)
    out = _embed_sc(b * s)(xf, table)
    return out.reshape(b, s, D_MODEL)
